# baseline (speedup 1.0000x reference)
import functools
import math

import jax
import jax.numpy as jnp
from jax import lax
from jax.experimental import pallas as pl
from jax.experimental.pallas import tpu as pltpu
from jax.experimental.pallas import tpu_sc as plsc

VOCAB = 100000
D = 64
SCALE = math.sqrt(D)

NC = 2
NS = 16
NW = NC * NS

B_SEQ = 4096
T_SEQ = 200
B_TOK = B_SEQ * T_SEQ
PER_W = B_TOK // NW
IDX_ROWS_PER_W = PER_W // 128
CROWS = 2
CH = CROWS * 128
NCH = PER_W // CH
PREP_COLS = 4096
PREP_GRID = 25
TBL_ROWS = PREP_GRID * PREP_COLS


def _prep_body(x_ref, o_ref):
    x = x_ref[...]
    ya = x[:, : PREP_COLS // 2].T
    yb = x[:, PREP_COLS // 2 :].T
    o_ref[...] = jnp.concatenate([ya, yb], axis=1) * SCALE


def _prep_table(table):
    tt = table.T
    out = pl.pallas_call(
        _prep_body,
        grid=(PREP_GRID,),
        in_specs=[pl.BlockSpec((D, PREP_COLS), lambda i: (0, i))],
        out_specs=pl.BlockSpec((PREP_COLS // 2, 128), lambda i: (i, 0)),
        out_shape=jax.ShapeDtypeStruct((TBL_ROWS // 2, 128), jnp.float32),
    )(tt)
    return out.reshape(TBL_ROWS, D)


def _gather_body(tok_hbm, table_hbm, out_hbm,
                 idx_v, r0, r1, t0, t1, g0, g1, s0, s1):
    wid = lax.axis_index("s") * NC + lax.axis_index("c")
    m_base = wid * PER_W

    def gather_descs(h, rbuf, gsem):
        return [
            pltpu.make_async_copy(
                table_hbm.at[idx_v.at[h * CROWS + k]],
                rbuf.at[pl.ds(k * 128, 128)],
                gsem,
            )
            for k in range(CROWS)
        ]

    def scatter_descs(h, tbuf, ssem):
        m0 = m_base + h * CH
        t = m0 // B_SEQ
        c0 = (m0 % B_SEQ) // 128
        return [
            pltpu.make_async_copy(
                tbuf.at[:, :, pl.ds(128 * cc, 128)],
                out_hbm.at[t, :, c0 + cc, :, :],
                ssem,
            )
            for cc in range(CH // 128)
        ]

    def start(ds):
        for d in ds:
            d.start()

    def wait(ds):
        for d in ds:
            d.wait()

    def transpose_scale(rbuf, tbuf):
        iot = lax.iota(jnp.int32, 16)
        d0s = [jnp.right_shift(16 * k + iot, 3) for k in range(D // 16)]
        drs = [jnp.bitwise_and(16 * k + iot, 7) for k in range(D // 16)]

        @plsc.parallel_loop(0, CH, unroll=4)
        def _(i):
            cidx = jnp.full((16,), 0, jnp.int32) + i
            for k in range(D // 16):
                vals = rbuf[i, pl.ds(16 * k, 16)]
                plsc.store_scatter(tbuf, [d0s[k], drs[k], cidx], vals)

    pltpu.sync_copy(tok_hbm.at[wid], idx_v)

    start(gather_descs(0, r0, g0))
    start(gather_descs(1, r1, g1))

    def step(p, carry):
        h = 2 * p
        wait(gather_descs(h, r0, g0))

        @pl.when(h >= 2)
        def _():
            wait(scatter_descs(h - 2, t0, s0))

        transpose_scale(r0, t0)
        start(gather_descs(h + 2, r0, g0))
        start(scatter_descs(h, t0, s0))
        wait(gather_descs(h + 1, r1, g1))

        @pl.when(h >= 2)
        def _():
            wait(scatter_descs(h - 1, t1, s1))

        transpose_scale(r1, t1)

        @pl.when(h + 3 < NCH)
        def _():
            start(gather_descs(h + 3, r1, g1))

        start(scatter_descs(h + 1, t1, s1))
        return carry

    lax.fori_loop(0, NCH // 2 - 1, step, 0)

    h = NCH - 2
    wait(gather_descs(h, r0, g0))
    wait(scatter_descs(h - 2, t0, s0))
    transpose_scale(r0, t0)
    start(scatter_descs(h, t0, s0))
    wait(gather_descs(h + 1, r1, g1))
    wait(scatter_descs(h - 1, t1, s1))
    transpose_scale(r1, t1)
    start(scatter_descs(h + 1, t1, s1))
    wait(scatter_descs(h, t0, s0))
    wait(scatter_descs(h + 1, t1, s1))


def _sc_gather(tok3d, table):
    kern = functools.partial(
        pl.kernel,
        mesh=plsc.VectorSubcoreMesh(core_axis_name="c", subcore_axis_name="s"),
        out_type=jax.ShapeDtypeStruct((T_SEQ, 8, 32, 8, 128), jnp.float32),
        scratch_types=[
            pltpu.VMEM((IDX_ROWS_PER_W, 128), jnp.int32),
            pltpu.VMEM((CH, D), jnp.float32),
            pltpu.VMEM((CH, D), jnp.float32),
            pltpu.VMEM((8, 8, CH + 1), jnp.float32),
            pltpu.VMEM((8, 8, CH + 1), jnp.float32),
            pltpu.SemaphoreType.DMA,
            pltpu.SemaphoreType.DMA,
            pltpu.SemaphoreType.DMA,
            pltpu.SemaphoreType.DMA,
        ],
        compiler_params=pltpu.CompilerParams(use_tc_tiling_on_sc=False, needs_layout_passes=False),
    )(_gather_body)
    return kern(tok3d, table)


def kernel(tokens, embedding_table):
    half = PREP_COLS // 2
    shift = half.bit_length() - 1
    t32 = tokens.astype(jnp.int32).T
    g = (
        jnp.bitwise_and(t32, ~(PREP_COLS - 1))
        + 2 * jnp.bitwise_and(t32, half - 1)
        + jnp.bitwise_and(jnp.right_shift(t32, shift), 1)
    )
    tok3d = g.reshape(NW, IDX_ROWS_PER_W, 128)
    out5 = _sc_gather(tok3d, _prep_table(embedding_table))
    outp = out5.transpose(0, 1, 3, 2, 4).reshape(T_SEQ, D, B_SEQ)
    return jnp.transpose(outp, (2, 0, 1))


# speedup vs baseline: 1.0064x; 1.0064x over previous
import functools
import math

import jax
import jax.numpy as jnp
from jax import lax
from jax.experimental import pallas as pl
from jax.experimental.pallas import tpu as pltpu
from jax.experimental.pallas import tpu_sc as plsc

VOCAB = 100000
D = 64
SCALE = math.sqrt(D)

NC = 2
NS = 16
NW = NC * NS

B_SEQ = 4096
T_SEQ = 200
B_TOK = B_SEQ * T_SEQ
PER_W = B_TOK // NW
IDX_ROWS_PER_W = PER_W // 128
CROWS = 1
CH = CROWS * 128
NCH = PER_W // CH
NB = 4
PREP_COLS = 4096
PREP_GRID = 25
TBL_ROWS = PREP_GRID * PREP_COLS


def _prep_body(x_ref, o_ref):
    x = x_ref[...]
    ya = x[:, : PREP_COLS // 2].T
    yb = x[:, PREP_COLS // 2 :].T
    o_ref[...] = jnp.concatenate([ya, yb], axis=1) * SCALE


def _prep_table(table):
    tt = table.T
    out = pl.pallas_call(
        _prep_body,
        grid=(PREP_GRID,),
        in_specs=[pl.BlockSpec((D, PREP_COLS), lambda i: (0, i))],
        out_specs=pl.BlockSpec((PREP_COLS // 2, 128), lambda i: (i, 0)),
        out_shape=jax.ShapeDtypeStruct((TBL_ROWS // 2, 128), jnp.float32),
    )(tt)
    return out.reshape(TBL_ROWS, D)


def _gather_body(tok_hbm, table_hbm, out_hbm, idx_v,
                 r0, r1, r2, r3, t0, t1, t2, t3,
                 g0, g1, g2, g3, s0, s1, s2, s3):
    wid = lax.axis_index("s") * NC + lax.axis_index("c")
    m_base = wid * PER_W

    def gather_descs(h, rbuf, gsem):
        return [
            pltpu.make_async_copy(
                table_hbm.at[idx_v.at[h * CROWS + k]],
                rbuf.at[pl.ds(k * 128, 128)],
                gsem,
            )
            for k in range(CROWS)
        ]

    def scatter_descs(h, tbuf, ssem):
        m0 = m_base + h * CH
        t = m0 // B_SEQ
        c0 = (m0 % B_SEQ) // 128
        return [
            pltpu.make_async_copy(
                tbuf.at[:, :, pl.ds(128 * cc, 128)],
                out_hbm.at[t, :, c0 + cc, :, :],
                ssem,
            )
            for cc in range(CH // 128)
        ]

    def start(ds):
        for d in ds:
            d.start()

    def wait(ds):
        for d in ds:
            d.wait()

    def transpose_scale(rbuf, tbuf):
        iot = lax.iota(jnp.int32, 16)
        d0s = [jnp.right_shift(16 * k + iot, 3) for k in range(D // 16)]
        drs = [jnp.bitwise_and(16 * k + iot, 7) for k in range(D // 16)]

        @plsc.parallel_loop(0, CH, unroll=4)
        def _(i):
            cidx = jnp.full((16,), 0, jnp.int32) + i
            for k in range(D // 16):
                vals = rbuf[i, pl.ds(16 * k, 16)]
                plsc.store_scatter(tbuf, [d0s[k], drs[k], cidx], vals)

    pltpu.sync_copy(tok_hbm.at[wid], idx_v)

    rb = (r0, r1, r2, r3)
    tb = (t0, t1, t2, t3)
    gs = (g0, g1, g2, g3)
    ss = (s0, s1, s2, s3)

    for b in range(NB):
        start(gather_descs(b, rb[b], gs[b]))

    def step(p, carry):
        for b in range(NB):
            h = NB * p + b
            wait(gather_descs(h, rb[b], gs[b]))

            @pl.when(p >= 1)
            def _():
                wait(scatter_descs(h - NB, tb[b], ss[b]))

            transpose_scale(rb[b], tb[b])

            @pl.when(p < NCH // NB - 1)
            def _():
                start(gather_descs(h + NB, rb[b], gs[b]))

            start(scatter_descs(h, tb[b], ss[b]))
        return carry

    lax.fori_loop(0, NCH // NB, step, 0)

    for b in range(NB):
        wait(scatter_descs(NCH - NB + b, tb[b], ss[b]))


def _sc_gather(tok3d, table):
    kern = functools.partial(
        pl.kernel,
        mesh=plsc.VectorSubcoreMesh(core_axis_name="c", subcore_axis_name="s"),
        out_type=jax.ShapeDtypeStruct((T_SEQ, 8, 32, 8, 128), jnp.float32),
        scratch_types=(
            [pltpu.VMEM((IDX_ROWS_PER_W, 128), jnp.int32)]
            + [pltpu.VMEM((CH, D), jnp.float32)] * 4
            + [pltpu.VMEM((8, 8, CH + 1), jnp.float32)] * 4
            + [pltpu.SemaphoreType.DMA] * 8
        ),
        compiler_params=pltpu.CompilerParams(use_tc_tiling_on_sc=False, needs_layout_passes=False),
    )(_gather_body)
    return kern(tok3d, table)


def kernel(tokens, embedding_table):
    half = PREP_COLS // 2
    shift = half.bit_length() - 1
    t32 = tokens.astype(jnp.int32).T
    g = (
        jnp.bitwise_and(t32, ~(PREP_COLS - 1))
        + 2 * jnp.bitwise_and(t32, half - 1)
        + jnp.bitwise_and(jnp.right_shift(t32, shift), 1)
    )
    tok3d = g.reshape(NW, IDX_ROWS_PER_W, 128)
    out5 = _sc_gather(tok3d, _prep_table(embedding_table))
    outp = out5.transpose(0, 1, 3, 2, 4).reshape(T_SEQ, D, B_SEQ)
    return jnp.transpose(outp, (2, 0, 1))
